# Initial kernel scaffold; baseline (speedup 1.0000x reference)
import jax
import jax.numpy as jnp
from jax.experimental import pallas as pl


def kernel(gene_input, smiles_input, gene_table, smiles_table, W1, b1, W2, b2, W3, b3):
    raise NotImplementedError("write your pallas kernel here")


# speedup vs baseline: 2.8615x; 2.8615x over previous
import functools

import jax
import jax.numpy as jnp
from jax import lax
from jax.experimental import pallas as pl
from jax.experimental.pallas import tpu as pltpu
from jax.experimental.pallas import tpu_sc as plsc

B = 4096
LG = 200
LS = 200
D = 128
NW = 32
CH = 128
N_LOOK = B * LG
PER_W = N_LOOK // NW
NCH = PER_W // CH

BM = 256
BK = 2560
NKG = (LG * D) // BK
NK = 2 * NKG


def _sc_gather(gidx, sidx, gtab, stab):
    mesh = plsc.VectorSubcoreMesh(core_axis_name="c", subcore_axis_name="s")

    @functools.partial(
        pl.kernel,
        out_type=[
            jax.ShapeDtypeStruct((N_LOOK, D), jnp.float32),
            jax.ShapeDtypeStruct((N_LOOK, D), jnp.float32),
        ],
        mesh=mesh,
        scratch_types=[
            pltpu.VMEM((NCH, CH), jnp.int32),
            pltpu.VMEM((CH, D), jnp.float32),
            pltpu.SemaphoreType.DMA,
        ],
    )
    def k(gidx_h, sidx_h, gtab_h, stab_h, gout_h, sout_h, idx_v, rows_v, sem):
        wid = lax.axis_index("s") * 2 + lax.axis_index("c")
        base = wid * PER_W

        def run(idx_h, tab_h, out_h):
            pltpu.sync_copy(idx_h.at[wid], idx_v)

            def body(j, carry):
                pltpu.async_copy(tab_h.at[idx_v.at[j]], rows_v, sem).wait()
                pltpu.sync_copy(rows_v, out_h.at[pl.ds(base + j * CH, CH)])
                return carry

            lax.fori_loop(0, NCH, body, 0)

        run(gidx_h, gtab_h, gout_h)
        run(sidx_h, stab_h, sout_h)

    return k(gidx, sidx, gtab, stab)


def _mlp(gg, gs, W1, b1, W2, b2, W3, b3):
    def body(gg_r, gs_r, w1_r, b1_r, w2_r, b2_r, w3_r, b3_r, out_r, acc_r):
        kk = pl.program_id(1)

        @pl.when(kk == 0)
        def _():
            acc_r[...] = jnp.zeros_like(acc_r)

        @pl.when(kk < NKG)
        def _():
            acc_r[...] += jnp.dot(gg_r[...], w1_r[...],
                                  preferred_element_type=jnp.float32)

        @pl.when(kk >= NKG)
        def _():
            acc_r[...] += jnp.dot(gs_r[...], w1_r[...],
                                  preferred_element_type=jnp.float32)

        @pl.when(kk == NK - 1)
        def _():
            h = jnp.maximum(acc_r[...] + b1_r[...], 0.0)
            h = jnp.maximum(
                jnp.dot(h, w2_r[...], preferred_element_type=jnp.float32)
                + b2_r[...], 0.0)
            z = jnp.dot(h, w3_r[...], preferred_element_type=jnp.float32) + b3_r[...]
            out_r[...] = jax.nn.sigmoid(z)

    return pl.pallas_call(
        body,
        grid=(B // BM, NK),
        in_specs=[
            pl.BlockSpec((BM, BK), lambda i, k: (i, jnp.minimum(k, NKG - 1))),
            pl.BlockSpec((BM, BK), lambda i, k: (i, jnp.maximum(k - NKG, 0))),
            pl.BlockSpec((BK, 64), lambda i, k: (k, 0)),
            pl.BlockSpec((1, 64), lambda i, k: (0, 0)),
            pl.BlockSpec((64, 32), lambda i, k: (0, 0)),
            pl.BlockSpec((1, 32), lambda i, k: (0, 0)),
            pl.BlockSpec((32, 1), lambda i, k: (0, 0)),
            pl.BlockSpec((1, 1), lambda i, k: (0, 0)),
        ],
        out_specs=pl.BlockSpec((BM, 1), lambda i, k: (i, 0)),
        out_shape=jax.ShapeDtypeStruct((B, 1), jnp.float32),
        scratch_shapes=[pltpu.VMEM((BM, 64), jnp.float32)],
        compiler_params=pltpu.CompilerParams(
            dimension_semantics=("parallel", "arbitrary")),
    )(gg, gs, W1, b1, W2, b2, W3, b3)


def kernel(gene_input, smiles_input, gene_table, smiles_table,
           W1, b1, W2, b2, W3, b3):
    gidx = gene_input.reshape(NW, NCH, CH)
    sidx = smiles_input.reshape(NW, NCH, CH)
    gg, gs = _sc_gather(gidx, sidx, gene_table, smiles_table)
    gg = gg.reshape(B, LG * D)
    gs = gs.reshape(B, LS * D)
    return _mlp(gg, gs, W1, b1.reshape(1, 64), W2, b2.reshape(1, 32),
                W3, b3.reshape(1, 1))
